# baseline (speedup 1.0000x reference)
import jax
import jax.numpy as jnp
import numpy as np
from jax import lax
from jax.experimental import pallas as pl
from jax.experimental.pallas import tpu as pltpu

_B, _NG, _HS, _WS = 4, 8, 128, 128
_STRIDE = 4.0
_ETA = 3.0
_KCAP = 64
_PW = 1.2
_BIG = 1048576.0


def _softplus(x):
    return jnp.maximum(x, 0.0) + jnp.log1p(jnp.exp(-jnp.abs(x)))


def _seg_dist_sq(px, py, x1, y1, x2, y2):
    vx = x2 - x1
    vy = y2 - y1
    wx = px - x1
    wy = py - y1
    vv = vx * vx + vy * vy + 1e-9
    t = jnp.clip((wx * vx + wy * vy) / vv, 0.0, 1.0)
    dx = wx - t * vx
    dy = wy - t * vy
    return dx * dx + dy * dy + 1e-12


def _loss_kernel(gt_ref, reg_ref, obj_ref, cls_ref, out_ref):
    row = lax.broadcasted_iota(jnp.int32, (_HS, _WS), 0).astype(jnp.float32)
    col = lax.broadcasted_iota(jnp.int32, (_HS, _WS), 1).astype(jnp.float32)
    py = (row + 0.5) * _STRIDE
    px = (col + 0.5) * _STRIDE
    lin = row * jnp.float32(_WS) + col

    keys_list = []
    for b in range(_B):
        for j in range(_NG):
            Ax = gt_ref[b, j, 0, 0]
            Ay = gt_ref[b, j, 0, 1]
            Bx = gt_ref[b, j, 1, 0]
            By = gt_ref[b, j, 1, 1]
            Cx = gt_ref[b, j, 2, 0]
            Cy = gt_ref[b, j, 2, 1]
            d1 = (px - Bx) * (Ay - By) - (Ax - Bx) * (py - By)
            d2 = (px - Cx) * (By - Cy) - (Bx - Cx) * (py - Cy)
            d3 = (px - Ax) * (Cy - Ay) - (Cx - Ax) * (py - Ay)
            has_neg = (d1 < 0) | (d2 < 0) | (d3 < 0)
            has_pos = (d1 > 0) | (d2 > 0) | (d3 > 0)
            inside = ~(has_neg & has_pos)
            dsq = jnp.minimum(
                _seg_dist_sq(px, py, Ax, Ay, Bx, By),
                jnp.minimum(_seg_dist_sq(px, py, Bx, By, Cx, Cy),
                            _seg_dist_sq(px, py, Cx, Cy, Ax, Ay)))
            pos = inside | (dsq <= _ETA * _ETA)
            keys_list.append(jnp.where(pos, dsq, _BIG))
    keys0 = jnp.stack(keys_list)
    lin3 = jnp.broadcast_to(lin[None], (_B * _NG, _HS, _WS))

    nmap = _B * _NG
    ibits = lax.bitcast_convert_type(keys0, jnp.int32)
    big_bits = np.float32(_BIG).view(np.int32).item()
    min_bits = np.float32(1e-12).view(np.int32).item()

    def _count_le(mask_f32):
        return jnp.sum(jnp.sum(mask_f32, axis=1), axis=1)

    npix = _count_le((ibits < big_bits).astype(jnp.float32))
    small = npix <= jnp.float32(_KCAP)

    colmin = jnp.min(ibits, axis=1)
    m0 = jnp.min(colmin, axis=1)

    def cm_body(_, carry):
        lo, hi = carry
        mid = lo + lax.shift_right_logical(hi - lo, 1)
        cnt = jnp.sum((colmin <= mid[:, None]).astype(jnp.float32), axis=1)
        ge_k = cnt >= jnp.float32(_KCAP)
        return jnp.where(ge_k, lo, mid + 1), jnp.where(ge_k, mid, hi)

    c64, _ = lax.fori_loop(0, 29, cm_body,
                           (jnp.full((nmap,), min_bits, jnp.int32),
                            jnp.full((nmap,), big_bits, jnp.int32)))

    small_i = small.astype(jnp.int32)

    def bs_cond(carry):
        i, lo, hi, tsel, done = carry
        return jnp.logical_and(i < 29, jnp.min(done) == 0)

    def bs_body(carry):
        i, lo, hi, tsel, done = carry
        mid = lo + lax.shift_right_logical(hi - lo, 1)
        cnt = _count_le((ibits <= mid[:, None, None]).astype(jnp.float32))
        hit = (cnt == jnp.float32(_KCAP)) & (done == 0)
        tsel = jnp.where(hit, mid, tsel)
        done = jnp.where(hit, 1, done)
        ge_k = cnt >= jnp.float32(_KCAP)
        return (i + 1, jnp.where(ge_k, lo, mid + 1),
                jnp.where(ge_k, mid, hi), tsel, done)

    lo0 = m0
    hi0 = c64
    _, lo_f, _, tsel, done = lax.while_loop(
        bs_cond, bs_body,
        (jnp.int32(0), lo0, hi0, jnp.zeros((nmap,), jnp.int32), small_i))
    slt = jnp.where(small, big_bits,
                    jnp.where(done == 1, tsel + 1, lo_f))
    t3 = slt[:, None, None]
    sel_lt = (ibits < t3).astype(jnp.float32)
    cnt_lt = _count_le(sel_lt)
    k_extra = jnp.where(done == 1, jnp.float32(0.0),
                        jnp.float32(_KCAP) - cnt_lt)
    lin3i = lin3.astype(jnp.int32)
    tie = ((ibits == t3) & (done == 0)[:, None, None]).astype(jnp.float32)
    cnt_tie = _count_le(tie)

    def tie_all():
        return jnp.full((nmap,), _HS * _WS, jnp.int32)

    def tie_search():
        def tie_bs_body(_, carry):
            lo, hi = carry
            mid = lo + lax.shift_right_logical(hi - lo, 1)
            cnt = _count_le(
                tie * (lin3i <= mid[:, None, None]).astype(jnp.float32))
            ge_k = cnt >= k_extra
            return jnp.where(ge_k, lo, mid + 1), jnp.where(ge_k, mid, hi)

        lthr, _ = lax.fori_loop(
            0, 14, tie_bs_body,
            (jnp.zeros((nmap,), jnp.int32),
             jnp.full((nmap,), _HS * _WS - 1, jnp.int32)))
        return lthr

    quick = jnp.all((k_extra == 0.0) | (k_extra == cnt_tie))
    lthr = lax.cond(quick, tie_all, tie_search)
    tie_on = (k_extra > 0)[:, None, None].astype(jnp.float32)
    selmask = sel_lt + tie * tie_on * (
        lin3i <= lthr[:, None, None]).astype(jnp.float32)

    reg_sum = jnp.float32(0.0)
    obj_sum = jnp.float32(0.0)
    cls_sum = jnp.float32(0.0)
    pos_now_sum = jnp.float32(0.0)
    nsel_sum = jnp.float32(0.0)
    row5 = row + 0.5
    col5 = col + 0.5
    for b in range(_B):
        smb = selmask[b * _NG:(b + 1) * _NG]
        cnt = jnp.sum(smb, axis=0)
        obj_t = jnp.minimum(cnt, 1.0)
        nsel_sum = nsel_sum + jnp.sum(cnt)
        pos_now_sum = pos_now_sum + jnp.sum(obj_t)

        xo = obj_ref[b, 0]
        obj_sum = obj_sum + jnp.sum(
            (1.0 - obj_t) * xo + (1.0 + (_PW - 1.0) * obj_t) * _softplus(-xo))

        cls_sum = cls_sum + jnp.sum(cnt * _softplus(-cls_ref[b, 0]))

        o = [jnp.clip(reg_ref[b, c], -64.0, 64.0) for c in range(6)]
        for j in range(_NG):
            gx = [gt_ref[b, j, p, 0] * (1.0 / _STRIDE) - col5 for p in range(3)]
            gy = [gt_ref[b, j, p, 1] * (1.0 / _STRIDE) - row5 for p in range(3)]
            p0 = (o[0] - gx[0]) ** 2 + (o[1] - gy[0]) ** 2
            d11 = jnp.sqrt((o[2] - gx[1]) ** 2 + (o[3] - gy[1]) ** 2)
            d12 = jnp.sqrt((o[2] - gx[2]) ** 2 + (o[3] - gy[2]) ** 2)
            d21 = jnp.sqrt((o[4] - gx[1]) ** 2 + (o[5] - gy[1]) ** 2)
            d22 = jnp.sqrt((o[4] - gx[2]) ** 2 + (o[5] - gy[2]) ** 2)
            cd = (jnp.minimum(d11, d12) + jnp.minimum(d21, d22)
                  + jnp.minimum(d11, d21) + jnp.minimum(d12, d22))
            reg_sum = reg_sum + jnp.sum(smb[j] * (p0 + cd))

    li = lax.broadcasted_iota(jnp.int32, (1, 128), 1)
    out = jnp.where(li == 0, reg_sum,
          jnp.where(li == 1, obj_sum,
          jnp.where(li == 2, cls_sum,
          jnp.where(li == 3, pos_now_sum,
          jnp.where(li == 4, nsel_sum, 0.0)))))
    out_ref[...] = out


def _run(gt, pred_reg, pred_obj, pred_cls, interpret=False):
    return pl.pallas_call(
        _loss_kernel,
        out_shape=jax.ShapeDtypeStruct((1, 128), jnp.float32),
        in_specs=[
            pl.BlockSpec(memory_space=pltpu.SMEM),
            pl.BlockSpec(memory_space=pltpu.VMEM),
            pl.BlockSpec(memory_space=pltpu.VMEM),
            pl.BlockSpec(memory_space=pltpu.VMEM),
        ],
        out_specs=pl.BlockSpec(memory_space=pltpu.VMEM),
        interpret=interpret,
    )(gt, pred_reg, pred_obj, pred_cls)


def kernel(pred_reg, pred_obj, pred_cls, gt_points):
    gt = jnp.asarray(gt_points, jnp.float32)
    res = _run(gt, pred_reg, pred_obj, pred_cls)
    reg = res[0, 0]
    obj = res[0, 1]
    cls = res[0, 2]
    pos_now = res[0, 3]
    nsel = res[0, 4]
    pos_eps = jnp.maximum(1.0, nsel)
    neg = jnp.maximum(1.0, jnp.float32(_B * _HS * _WS) - pos_now)
    return reg / pos_eps + obj / (pos_eps + neg) + cls / pos_eps


# speedup vs baseline: 1.0447x; 1.0447x over previous
import jax
import jax.numpy as jnp
import numpy as np
from jax import lax
from jax.experimental import pallas as pl
from jax.experimental.pallas import tpu as pltpu

_B, _NG, _HS, _WS = 4, 8, 128, 128
_STRIDE = 4.0
_ETA = 3.0
_KCAP = 64
_PW = 1.2
_BIG = 1048576.0


def _softplus(x):
    return jnp.maximum(x, 0.0) + jnp.log1p(jnp.exp(-jnp.abs(x)))


def _seg_dist_sq(px, py, x1, y1, x2, y2):
    vx = x2 - x1
    vy = y2 - y1
    wx = px - x1
    wy = py - y1
    vv = vx * vx + vy * vy + 1e-9
    t = jnp.clip((wx * vx + wy * vy) / vv, 0.0, 1.0)
    dx = wx - t * vx
    dy = wy - t * vy
    return dx * dx + dy * dy + 1e-12


def _loss_kernel(gt_ref, reg_ref, obj_ref, cls_ref, out_ref):
    row = lax.broadcasted_iota(jnp.int32, (_HS, _WS), 0).astype(jnp.float32)
    col = lax.broadcasted_iota(jnp.int32, (_HS, _WS), 1).astype(jnp.float32)
    py = (row + 0.5) * _STRIDE
    px = (col + 0.5) * _STRIDE
    lin = row * jnp.float32(_WS) + col

    keys_list = []
    for b in range(_B):
        for j in range(_NG):
            Ax = gt_ref[b, j, 0, 0]
            Ay = gt_ref[b, j, 0, 1]
            Bx = gt_ref[b, j, 1, 0]
            By = gt_ref[b, j, 1, 1]
            Cx = gt_ref[b, j, 2, 0]
            Cy = gt_ref[b, j, 2, 1]
            d1 = (px - Bx) * (Ay - By) - (Ax - Bx) * (py - By)
            d2 = (px - Cx) * (By - Cy) - (Bx - Cx) * (py - Cy)
            d3 = (px - Ax) * (Cy - Ay) - (Cx - Ax) * (py - Ay)
            has_neg = (d1 < 0) | (d2 < 0) | (d3 < 0)
            has_pos = (d1 > 0) | (d2 > 0) | (d3 > 0)
            inside = ~(has_neg & has_pos)
            dsq = jnp.minimum(
                _seg_dist_sq(px, py, Ax, Ay, Bx, By),
                jnp.minimum(_seg_dist_sq(px, py, Bx, By, Cx, Cy),
                            _seg_dist_sq(px, py, Cx, Cy, Ax, Ay)))
            pos = inside | (dsq <= _ETA * _ETA)
            keys_list.append(jnp.where(pos, dsq, _BIG))
    keys0 = jnp.stack(keys_list)
    lin3 = jnp.broadcast_to(lin[None], (_B * _NG, _HS, _WS))

    nmap = _B * _NG
    ibits = lax.bitcast_convert_type(keys0, jnp.int32)
    big_bits = np.float32(_BIG).view(np.int32).item()
    min_bits = np.float32(1e-12).view(np.int32).item()

    def _count_le(mask_f32):
        return jnp.sum(jnp.sum(mask_f32, axis=1), axis=1)

    npix = _count_le((ibits < big_bits).astype(jnp.float32))
    small = npix <= jnp.float32(_KCAP)

    small_i = small.astype(jnp.int32)
    kf = jnp.float32(_KCAP)

    def bs_cond(carry):
        i, lo, hi, tsel, done = carry
        return jnp.logical_and(
            i < 29, jnp.min(jnp.maximum(done, (lo >= hi).astype(jnp.int32)))
            == 0)

    def bs_body(carry):
        i, lo, hi, tsel, done = carry
        q = lax.shift_right_logical(hi - lo, 2)
        p1 = lo + q
        p2 = p1 + q
        p3 = p2 + q
        p4 = jnp.maximum(hi - 1, lo)
        c1 = _count_le((ibits <= p1[:, None, None]).astype(jnp.float32))
        c2 = _count_le((ibits <= p2[:, None, None]).astype(jnp.float32))
        c3 = _count_le((ibits <= p3[:, None, None]).astype(jnp.float32))
        c4 = _count_le((ibits <= p4[:, None, None]).astype(jnp.float32))
        live = done == 0
        hit = ((c1 == kf) | (c2 == kf) | (c3 == kf) | (c4 == kf)) & live
        tnew = jnp.where(c1 == kf, p1,
               jnp.where(c2 == kf, p2,
               jnp.where(c3 == kf, p3, p4)))
        tsel = jnp.where(hit, tnew, tsel)
        done = jnp.where(hit, 1, done)
        lo_n = jnp.where(c1 >= kf, lo,
               jnp.where(c2 >= kf, p1 + 1,
               jnp.where(c3 >= kf, p2 + 1,
               jnp.where(c4 >= kf, p3 + 1, p4 + 1))))
        hi_n = jnp.where(c1 >= kf, p1,
               jnp.where(c2 >= kf, p2,
               jnp.where(c3 >= kf, p3,
               jnp.where(c4 >= kf, p4, hi))))
        return i + 1, lo_n, hi_n, tsel, done

    lo0 = jnp.full((nmap,), min_bits, jnp.int32)
    hi0 = jnp.full((nmap,), big_bits, jnp.int32)
    _, lo_f, _, tsel, done = lax.while_loop(
        bs_cond, bs_body,
        (jnp.int32(0), lo0, hi0, jnp.zeros((nmap,), jnp.int32), small_i))
    slt = jnp.where(small, big_bits,
                    jnp.where(done == 1, tsel + 1, lo_f))
    t3 = slt[:, None, None]
    sel_lt = (ibits < t3).astype(jnp.float32)
    cnt_lt = _count_le(sel_lt)
    k_extra = jnp.where(done == 1, jnp.float32(0.0),
                        jnp.float32(_KCAP) - cnt_lt)
    lin3i = lin3.astype(jnp.int32)
    tie = ((ibits == t3) & (done == 0)[:, None, None]).astype(jnp.float32)
    cnt_tie = _count_le(tie)

    def tie_all():
        return jnp.full((nmap,), _HS * _WS, jnp.int32)

    def tie_search():
        def tie_bs_body(_, carry):
            lo, hi = carry
            mid = lo + lax.shift_right_logical(hi - lo, 1)
            cnt = _count_le(
                tie * (lin3i <= mid[:, None, None]).astype(jnp.float32))
            ge_k = cnt >= k_extra
            return jnp.where(ge_k, lo, mid + 1), jnp.where(ge_k, mid, hi)

        lthr, _ = lax.fori_loop(
            0, 14, tie_bs_body,
            (jnp.zeros((nmap,), jnp.int32),
             jnp.full((nmap,), _HS * _WS - 1, jnp.int32)))
        return lthr

    quick = jnp.all((k_extra == 0.0) | (k_extra == cnt_tie))
    lthr = lax.cond(quick, tie_all, tie_search)
    tie_on = (k_extra > 0)[:, None, None].astype(jnp.float32)
    selmask = sel_lt + tie * tie_on * (
        lin3i <= lthr[:, None, None]).astype(jnp.float32)

    reg_sum = jnp.float32(0.0)
    obj_sum = jnp.float32(0.0)
    cls_sum = jnp.float32(0.0)
    pos_now_sum = jnp.float32(0.0)
    nsel_sum = jnp.float32(0.0)
    row5 = row + 0.5
    col5 = col + 0.5
    for b in range(_B):
        smb = selmask[b * _NG:(b + 1) * _NG]
        cnt = jnp.sum(smb, axis=0)
        obj_t = jnp.minimum(cnt, 1.0)
        nsel_sum = nsel_sum + jnp.sum(cnt)
        pos_now_sum = pos_now_sum + jnp.sum(obj_t)

        xo = obj_ref[b, 0]
        obj_sum = obj_sum + jnp.sum(
            (1.0 - obj_t) * xo + (1.0 + (_PW - 1.0) * obj_t) * _softplus(-xo))

        cls_sum = cls_sum + jnp.sum(cnt * _softplus(-cls_ref[b, 0]))

        o = [jnp.clip(reg_ref[b, c], -64.0, 64.0) for c in range(6)]
        for j in range(_NG):
            gx = [gt_ref[b, j, p, 0] * (1.0 / _STRIDE) - col5 for p in range(3)]
            gy = [gt_ref[b, j, p, 1] * (1.0 / _STRIDE) - row5 for p in range(3)]
            p0 = (o[0] - gx[0]) ** 2 + (o[1] - gy[0]) ** 2
            d11 = jnp.sqrt((o[2] - gx[1]) ** 2 + (o[3] - gy[1]) ** 2)
            d12 = jnp.sqrt((o[2] - gx[2]) ** 2 + (o[3] - gy[2]) ** 2)
            d21 = jnp.sqrt((o[4] - gx[1]) ** 2 + (o[5] - gy[1]) ** 2)
            d22 = jnp.sqrt((o[4] - gx[2]) ** 2 + (o[5] - gy[2]) ** 2)
            cd = (jnp.minimum(d11, d12) + jnp.minimum(d21, d22)
                  + jnp.minimum(d11, d21) + jnp.minimum(d12, d22))
            reg_sum = reg_sum + jnp.sum(smb[j] * (p0 + cd))

    li = lax.broadcasted_iota(jnp.int32, (1, 128), 1)
    out = jnp.where(li == 0, reg_sum,
          jnp.where(li == 1, obj_sum,
          jnp.where(li == 2, cls_sum,
          jnp.where(li == 3, pos_now_sum,
          jnp.where(li == 4, nsel_sum, 0.0)))))
    out_ref[...] = out


def _run(gt, pred_reg, pred_obj, pred_cls, interpret=False):
    return pl.pallas_call(
        _loss_kernel,
        out_shape=jax.ShapeDtypeStruct((1, 128), jnp.float32),
        in_specs=[
            pl.BlockSpec(memory_space=pltpu.SMEM),
            pl.BlockSpec(memory_space=pltpu.VMEM),
            pl.BlockSpec(memory_space=pltpu.VMEM),
            pl.BlockSpec(memory_space=pltpu.VMEM),
        ],
        out_specs=pl.BlockSpec(memory_space=pltpu.VMEM),
        interpret=interpret,
    )(gt, pred_reg, pred_obj, pred_cls)


def kernel(pred_reg, pred_obj, pred_cls, gt_points):
    gt = jnp.asarray(gt_points, jnp.float32)
    res = _run(gt, pred_reg, pred_obj, pred_cls)
    reg = res[0, 0]
    obj = res[0, 1]
    cls = res[0, 2]
    pos_now = res[0, 3]
    nsel = res[0, 4]
    pos_eps = jnp.maximum(1.0, nsel)
    neg = jnp.maximum(1.0, jnp.float32(_B * _HS * _WS) - pos_now)
    return reg / pos_eps + obj / (pos_eps + neg) + cls / pos_eps


# speedup vs baseline: 1.0756x; 1.0297x over previous
import jax
import jax.numpy as jnp
import numpy as np
from jax import lax
from jax.experimental import pallas as pl
from jax.experimental.pallas import tpu as pltpu

_B, _NG, _HS, _WS = 4, 8, 128, 128
_STRIDE = 4.0
_ETA = 3.0
_KCAP = 64
_PW = 1.2
_BIG = 1048576.0


def _softplus(x):
    return jnp.maximum(x, 0.0) + jnp.log1p(jnp.exp(-jnp.abs(x)))


def _seg_dist_sq(px, py, x1, y1, x2, y2):
    vx = x2 - x1
    vy = y2 - y1
    wx = px - x1
    wy = py - y1
    vv = vx * vx + vy * vy + 1e-9
    t = jnp.clip((wx * vx + wy * vy) / vv, 0.0, 1.0)
    dx = wx - t * vx
    dy = wy - t * vy
    return dx * dx + dy * dy + 1e-12


def _loss_kernel(gt_ref, reg_ref, obj_ref, cls_ref, out_ref):
    row = lax.broadcasted_iota(jnp.int32, (_HS, _WS), 0).astype(jnp.float32)
    col = lax.broadcasted_iota(jnp.int32, (_HS, _WS), 1).astype(jnp.float32)
    py = (row + 0.5) * _STRIDE
    px = (col + 0.5) * _STRIDE

    keys_list = []
    for b in range(_B):
        for j in range(_NG):
            Ax = gt_ref[b, j, 0, 0]
            Ay = gt_ref[b, j, 0, 1]
            Bx = gt_ref[b, j, 1, 0]
            By = gt_ref[b, j, 1, 1]
            Cx = gt_ref[b, j, 2, 0]
            Cy = gt_ref[b, j, 2, 1]
            d1 = (px - Bx) * (Ay - By) - (Ax - Bx) * (py - By)
            d2 = (px - Cx) * (By - Cy) - (Bx - Cx) * (py - Cy)
            d3 = (px - Ax) * (Cy - Ay) - (Cx - Ax) * (py - Ay)
            has_neg = (d1 < 0) | (d2 < 0) | (d3 < 0)
            has_pos = (d1 > 0) | (d2 > 0) | (d3 > 0)
            inside = ~(has_neg & has_pos)
            dsq = jnp.minimum(
                _seg_dist_sq(px, py, Ax, Ay, Bx, By),
                jnp.minimum(_seg_dist_sq(px, py, Bx, By, Cx, Cy),
                            _seg_dist_sq(px, py, Cx, Cy, Ax, Ay)))
            pos = inside | (dsq <= _ETA * _ETA)
            keys_list.append(jnp.where(pos, dsq, _BIG))
    nmap = _B * _NG
    ib_list = [lax.bitcast_convert_type(k, jnp.int32) for k in keys_list]
    big_bits = np.float32(_BIG).view(np.int32).item()
    min_bits = np.float32(1e-12).view(np.int32).item()
    kf = jnp.float32(_KCAP)

    def _cnt2d(mask_bool):
        return jnp.sum(jnp.sum(mask_bool.astype(jnp.float32), axis=0))

    small = [ _cnt2d(ib < big_bits) <= kf for ib in ib_list ]

    def bs_cond(carry):
        i = carry[0]
        lo, hi, tsel, done = carry[1], carry[2], carry[3], carry[4]
        live = [ (done[m] == 0) & (lo[m] < hi[m]) for m in range(nmap) ]
        any_live = live[0]
        for m in range(1, nmap):
            any_live = any_live | live[m]
        return jnp.logical_and(i < 29, any_live)

    def bs_body(carry):
        i, lo, hi, tsel, done = carry
        lo_n, hi_n, t_n, d_n = [], [], [], []
        for m in range(nmap):
            mid = lo[m] + lax.shift_right_logical(hi[m] - lo[m], 1)
            cnt = _cnt2d(ib_list[m] <= mid)
            hit = (cnt == kf) & (done[m] == 0)
            t_n.append(jnp.where(hit, mid, tsel[m]))
            d_n.append(jnp.where(hit, 1, done[m]))
            ge_k = cnt >= kf
            lo_n.append(jnp.where(ge_k, lo[m], mid + 1))
            hi_n.append(jnp.where(ge_k, mid, hi[m]))
        return i + 1, tuple(lo_n), tuple(hi_n), tuple(t_n), tuple(d_n)

    init = (jnp.int32(0),
            tuple(jnp.int32(min_bits) for _ in range(nmap)),
            tuple(jnp.int32(big_bits) for _ in range(nmap)),
            tuple(jnp.int32(0) for _ in range(nmap)),
            tuple(s.astype(jnp.int32) for s in small))
    _, lo_f, _, tsel, done = lax.while_loop(bs_cond, bs_body, init)

    lini = lax.broadcasted_iota(jnp.int32, (_HS, _WS), 0) * _WS + \
        lax.broadcasted_iota(jnp.int32, (_HS, _WS), 1)
    slt = [jnp.where(small[m], big_bits,
                     jnp.where(done[m] == 1, tsel[m] + 1, lo_f[m]))
           for m in range(nmap)]
    sel_lt = [ib_list[m] < slt[m] for m in range(nmap)]
    tie = [(ib_list[m] == slt[m]) & (done[m] == 0) for m in range(nmap)]
    k_extra = [jnp.where(done[m] == 1, 0.0, kf - _cnt2d(sel_lt[m]))
               for m in range(nmap)]

    def tie_all():
        return tuple(jnp.int32(_HS * _WS) for _ in range(nmap))

    def tie_search():
        def tie_bs_body(_, carry):
            lo, hi = carry
            lo_n, hi_n = [], []
            for m in range(nmap):
                mid = lo[m] + lax.shift_right_logical(hi[m] - lo[m], 1)
                cnt = _cnt2d(tie[m] & (lini <= mid))
                ge_k = cnt >= k_extra[m]
                lo_n.append(jnp.where(ge_k, lo[m], mid + 1))
                hi_n.append(jnp.where(ge_k, mid, hi[m]))
            return tuple(lo_n), tuple(hi_n)

        lthr, _ = lax.fori_loop(
            0, 14, tie_bs_body,
            (tuple(jnp.int32(0) for _ in range(nmap)),
             tuple(jnp.int32(_HS * _WS - 1) for _ in range(nmap))))
        return lthr

    quickv = (k_extra[0] == 0.0) | (k_extra[0] == _cnt2d(tie[0]))
    for m in range(1, nmap):
        quickv = quickv & ((k_extra[m] == 0.0)
                           | (k_extra[m] == _cnt2d(tie[m])))
    lthr = lax.cond(quickv, tie_all, tie_search)
    selmask = [
        sel_lt[m].astype(jnp.float32)
        + (tie[m] & (k_extra[m] > 0) & (lini <= lthr[m])).astype(jnp.float32)
        for m in range(nmap)]

    reg_sum = jnp.float32(0.0)
    obj_sum = jnp.float32(0.0)
    cls_sum = jnp.float32(0.0)
    pos_now_sum = jnp.float32(0.0)
    nsel_sum = jnp.float32(0.0)
    row5 = row + 0.5
    col5 = col + 0.5
    for b in range(_B):
        smb = selmask[b * _NG:(b + 1) * _NG]
        cnt = smb[0]
        for j in range(1, _NG):
            cnt = cnt + smb[j]
        obj_t = jnp.minimum(cnt, 1.0)
        nsel_sum = nsel_sum + jnp.sum(cnt)
        pos_now_sum = pos_now_sum + jnp.sum(obj_t)

        xo = obj_ref[b, 0]
        obj_sum = obj_sum + jnp.sum(
            (1.0 - obj_t) * xo + (1.0 + (_PW - 1.0) * obj_t) * _softplus(-xo))

        cls_sum = cls_sum + jnp.sum(cnt * _softplus(-cls_ref[b, 0]))

        o = [jnp.clip(reg_ref[b, c], -64.0, 64.0) for c in range(6)]
        for j in range(_NG):
            gx = [gt_ref[b, j, p, 0] * (1.0 / _STRIDE) - col5 for p in range(3)]
            gy = [gt_ref[b, j, p, 1] * (1.0 / _STRIDE) - row5 for p in range(3)]
            p0 = (o[0] - gx[0]) ** 2 + (o[1] - gy[0]) ** 2
            d11 = jnp.sqrt((o[2] - gx[1]) ** 2 + (o[3] - gy[1]) ** 2)
            d12 = jnp.sqrt((o[2] - gx[2]) ** 2 + (o[3] - gy[2]) ** 2)
            d21 = jnp.sqrt((o[4] - gx[1]) ** 2 + (o[5] - gy[1]) ** 2)
            d22 = jnp.sqrt((o[4] - gx[2]) ** 2 + (o[5] - gy[2]) ** 2)
            cd = (jnp.minimum(d11, d12) + jnp.minimum(d21, d22)
                  + jnp.minimum(d11, d21) + jnp.minimum(d12, d22))
            reg_sum = reg_sum + jnp.sum(smb[j] * (p0 + cd))

    li = lax.broadcasted_iota(jnp.int32, (1, 128), 1)
    out = jnp.where(li == 0, reg_sum,
          jnp.where(li == 1, obj_sum,
          jnp.where(li == 2, cls_sum,
          jnp.where(li == 3, pos_now_sum,
          jnp.where(li == 4, nsel_sum, 0.0)))))
    out_ref[...] = out


def _run(gt, pred_reg, pred_obj, pred_cls, interpret=False):
    return pl.pallas_call(
        _loss_kernel,
        out_shape=jax.ShapeDtypeStruct((1, 128), jnp.float32),
        in_specs=[
            pl.BlockSpec(memory_space=pltpu.SMEM),
            pl.BlockSpec(memory_space=pltpu.VMEM),
            pl.BlockSpec(memory_space=pltpu.VMEM),
            pl.BlockSpec(memory_space=pltpu.VMEM),
        ],
        out_specs=pl.BlockSpec(memory_space=pltpu.VMEM),
        interpret=interpret,
    )(gt, pred_reg, pred_obj, pred_cls)


def kernel(pred_reg, pred_obj, pred_cls, gt_points):
    gt = jnp.asarray(gt_points, jnp.float32)
    res = _run(gt, pred_reg, pred_obj, pred_cls)
    reg = res[0, 0]
    obj = res[0, 1]
    cls = res[0, 2]
    pos_now = res[0, 3]
    nsel = res[0, 4]
    pos_eps = jnp.maximum(1.0, nsel)
    neg = jnp.maximum(1.0, jnp.float32(_B * _HS * _WS) - pos_now)
    return reg / pos_eps + obj / (pos_eps + neg) + cls / pos_eps


# speedup vs baseline: 1.3068x; 1.2149x over previous
import jax
import jax.numpy as jnp
import numpy as np
from jax import lax
from jax.experimental import pallas as pl
from jax.experimental.pallas import tpu as pltpu

_B, _NG, _HS, _WS = 4, 8, 128, 128
_STRIDE = 4.0
_ETA = 3.0
_KCAP = 64
_PW = 1.2
_BIG = 1048576.0


def _softplus(x):
    return jnp.maximum(x, 0.0) + jnp.log1p(jnp.exp(-jnp.abs(x)))


def _seg_dist_sq(wx, wy, x1, y1, x2, y2):
    vx = x2 - x1
    vy = y2 - y1
    vv = vx * vx + vy * vy + 1e-9
    t = jnp.clip((wx * vx + wy * vy) / vv, 0.0, 1.0)
    dx = wx - t * vx
    dy = wy - t * vy
    return dx * dx + dy * dy + 1e-12


def _loss_kernel(gt_ref, reg_ref, obj_ref, cls_ref, out_ref):
    row = lax.broadcasted_iota(jnp.int32, (_HS, _WS), 0).astype(jnp.float32)
    col = lax.broadcasted_iota(jnp.int32, (_HS, _WS), 1).astype(jnp.float32)
    py = (row + 0.5) * _STRIDE
    px = (col + 0.5) * _STRIDE

    keys_list = []
    for b in range(_B):
        for j in range(_NG):
            Ax = gt_ref[b, j, 0, 0]
            Ay = gt_ref[b, j, 0, 1]
            Bx = gt_ref[b, j, 1, 0]
            By = gt_ref[b, j, 1, 1]
            Cx = gt_ref[b, j, 2, 0]
            Cy = gt_ref[b, j, 2, 1]
            wxA = px - Ax
            wyA = py - Ay
            wxB = px - Bx
            wyB = py - By
            wxC = px - Cx
            wyC = py - Cy
            d1 = wxB * (Ay - By) - (Ax - Bx) * wyB
            d2 = wxC * (By - Cy) - (Bx - Cx) * wyC
            d3 = wxA * (Cy - Ay) - (Cx - Ax) * wyA
            has_neg = (d1 < 0) | (d2 < 0) | (d3 < 0)
            has_pos = (d1 > 0) | (d2 > 0) | (d3 > 0)
            inside = ~(has_neg & has_pos)
            dsq = jnp.minimum(
                _seg_dist_sq(wxA, wyA, Ax, Ay, Bx, By),
                jnp.minimum(_seg_dist_sq(wxB, wyB, Bx, By, Cx, Cy),
                            _seg_dist_sq(wxC, wyC, Cx, Cy, Ax, Ay)))
            pos = inside | (dsq <= _ETA * _ETA)
            keys_list.append(jnp.where(pos, dsq, _BIG))
    nmap = _B * _NG
    ib_list = [lax.bitcast_convert_type(k, jnp.int32) for k in keys_list]
    big_bits = np.float32(_BIG).view(np.int32).item()
    min_bits = np.float32(1e-12).view(np.int32).item()
    kf = jnp.float32(_KCAP)

    def _cnt2d(mask_bool):
        return jnp.sum(jnp.sum(mask_bool.astype(jnp.float32), axis=0))

    small = [ _cnt2d(ib < big_bits) <= kf for ib in ib_list ]

    def bs_cond(carry):
        i = carry[0]
        lo, hi, tsel, done = carry[1], carry[2], carry[3], carry[4]
        live = [ (done[m] == 0) & (lo[m] < hi[m]) for m in range(nmap) ]
        any_live = live[0]
        for m in range(1, nmap):
            any_live = any_live | live[m]
        return jnp.logical_and(i < 29, any_live)

    def bs_body(carry):
        i, lo, hi, tsel, done = carry
        lo_n, hi_n, t_n, d_n = [], [], [], []
        for m in range(nmap):
            mid = lo[m] + lax.shift_right_logical(hi[m] - lo[m], 1)
            cnt = _cnt2d(ib_list[m] <= mid)
            hit = (cnt == kf) & (done[m] == 0)
            t_n.append(jnp.where(hit, mid, tsel[m]))
            d_n.append(jnp.where(hit, 1, done[m]))
            ge_k = cnt >= kf
            lo_n.append(jnp.where(ge_k, lo[m], mid + 1))
            hi_n.append(jnp.where(ge_k, mid, hi[m]))
        return i + 1, tuple(lo_n), tuple(hi_n), tuple(t_n), tuple(d_n)

    init = (jnp.int32(0),
            tuple(jnp.int32(min_bits) for _ in range(nmap)),
            tuple(jnp.int32(big_bits) for _ in range(nmap)),
            tuple(jnp.int32(0) for _ in range(nmap)),
            tuple(s.astype(jnp.int32) for s in small))
    _, lo_f, _, tsel, done = lax.while_loop(bs_cond, bs_body, init)

    lini = lax.broadcasted_iota(jnp.int32, (_HS, _WS), 0) * _WS + \
        lax.broadcasted_iota(jnp.int32, (_HS, _WS), 1)
    slt = [jnp.where(small[m], big_bits,
                     jnp.where(done[m] == 1, tsel[m] + 1, lo_f[m]))
           for m in range(nmap)]
    sel_lt = [ib_list[m] < slt[m] for m in range(nmap)]
    tie = [(ib_list[m] == slt[m]) & (done[m] == 0) for m in range(nmap)]
    k_extra = [jnp.where(done[m] == 1, 0.0, kf - _cnt2d(sel_lt[m]))
               for m in range(nmap)]

    def tie_all():
        return tuple(jnp.int32(_HS * _WS) for _ in range(nmap))

    def tie_search():
        def tie_bs_body(_, carry):
            lo, hi = carry
            lo_n, hi_n = [], []
            for m in range(nmap):
                mid = lo[m] + lax.shift_right_logical(hi[m] - lo[m], 1)
                cnt = _cnt2d(tie[m] & (lini <= mid))
                ge_k = cnt >= k_extra[m]
                lo_n.append(jnp.where(ge_k, lo[m], mid + 1))
                hi_n.append(jnp.where(ge_k, mid, hi[m]))
            return tuple(lo_n), tuple(hi_n)

        lthr, _ = lax.fori_loop(
            0, 14, tie_bs_body,
            (tuple(jnp.int32(0) for _ in range(nmap)),
             tuple(jnp.int32(_HS * _WS - 1) for _ in range(nmap))))
        return lthr

    quickv = (k_extra[0] == 0.0) | (k_extra[0] == _cnt2d(tie[0]))
    for m in range(1, nmap):
        quickv = quickv & ((k_extra[m] == 0.0)
                           | (k_extra[m] == _cnt2d(tie[m])))
    lthr = lax.cond(quickv, tie_all, tie_search)
    selmask = [
        sel_lt[m].astype(jnp.float32)
        + (tie[m] & (k_extra[m] > 0) & (lini <= lthr[m])).astype(jnp.float32)
        for m in range(nmap)]

    reg_sum = jnp.float32(0.0)
    obj_sum = jnp.float32(0.0)
    cls_sum = jnp.float32(0.0)
    pos_now_sum = jnp.float32(0.0)
    nsel_sum = jnp.float32(0.0)
    row5 = row + 0.5
    col5 = col + 0.5
    for b in range(_B):
        smb = selmask[b * _NG:(b + 1) * _NG]
        cnt = smb[0]
        for j in range(1, _NG):
            cnt = cnt + smb[j]
        obj_t = jnp.minimum(cnt, 1.0)
        nsel_sum = nsel_sum + jnp.sum(cnt)
        pos_now_sum = pos_now_sum + jnp.sum(obj_t)

        xo = obj_ref[b, 0]
        obj_sum = obj_sum + jnp.sum(
            (1.0 - obj_t) * xo + (1.0 + (_PW - 1.0) * obj_t) * _softplus(-xo))

        cls_sum = cls_sum + jnp.sum(cnt * _softplus(-cls_ref[b, 0]))

        o = [jnp.clip(reg_ref[b, c], -64.0, 64.0) for c in range(6)]
        for j in range(_NG):
            gx = [gt_ref[b, j, p, 0] * (1.0 / _STRIDE) - col5 for p in range(3)]
            gy = [gt_ref[b, j, p, 1] * (1.0 / _STRIDE) - row5 for p in range(3)]
            p0 = (o[0] - gx[0]) ** 2 + (o[1] - gy[0]) ** 2
            d11 = jnp.sqrt((o[2] - gx[1]) ** 2 + (o[3] - gy[1]) ** 2)
            d12 = jnp.sqrt((o[2] - gx[2]) ** 2 + (o[3] - gy[2]) ** 2)
            d21 = jnp.sqrt((o[4] - gx[1]) ** 2 + (o[5] - gy[1]) ** 2)
            d22 = jnp.sqrt((o[4] - gx[2]) ** 2 + (o[5] - gy[2]) ** 2)
            cd = (jnp.minimum(d11, d12) + jnp.minimum(d21, d22)
                  + jnp.minimum(d11, d21) + jnp.minimum(d12, d22))
            reg_sum = reg_sum + jnp.sum(smb[j] * (p0 + cd))

    pos_eps = jnp.maximum(1.0, nsel_sum)
    neg = jnp.maximum(1.0, jnp.float32(_B * _HS * _WS) - pos_now_sum)
    final = (reg_sum / pos_eps + obj_sum / (pos_eps + neg)
             + cls_sum / pos_eps)
    li = lax.broadcasted_iota(jnp.int32, (1, 128), 1)
    out_ref[...] = jnp.where(li == 0, final, 0.0)


def _run(gt, pred_reg, pred_obj, pred_cls, interpret=False):
    return pl.pallas_call(
        _loss_kernel,
        out_shape=jax.ShapeDtypeStruct((1, 128), jnp.float32),
        in_specs=[
            pl.BlockSpec(memory_space=pltpu.SMEM),
            pl.BlockSpec(memory_space=pltpu.VMEM),
            pl.BlockSpec(memory_space=pltpu.VMEM),
            pl.BlockSpec(memory_space=pltpu.VMEM),
        ],
        out_specs=pl.BlockSpec(memory_space=pltpu.VMEM),
        interpret=interpret,
    )(gt, pred_reg, pred_obj, pred_cls)


def kernel(pred_reg, pred_obj, pred_cls, gt_points):
    gt = jnp.asarray(gt_points, jnp.float32)
    res = _run(gt, pred_reg, pred_obj, pred_cls)
    return res[0, 0]
